# Initial kernel scaffold; baseline (speedup 1.0000x reference)
import jax
import jax.numpy as jnp
from jax.experimental import pallas as pl


def kernel(node_indices, node_segment_ids, edge_sources, edge_targets, embedding, type_weights, type_biases, gru_gate_kernel, gru_gate_bias, gru_cand_kernel, gru_cand_bias):
    raise NotImplementedError("write your pallas kernel here")


# speedup vs baseline: 2.9941x; 2.9941x over previous
import functools

import jax
import jax.numpy as jnp
from jax import lax
from jax.experimental import pallas as pl
from jax.experimental.pallas import tpu as pltpu
from jax.experimental.pallas import tpu_sc as plsc

N_NODES = 10000
HIDDEN = 256
HALF = 128
VOCAB = 5000
N_TYPES = 4
EDGES_PER_TYPE = 40000
N_TOKENS = 20000
TIME_STEPS = [3, 3]

NC = 2
NS = 16
CHUNK = 128

NODE_P = 10112
ZROWS = NODE_P // NS

E_PAD = 161792
TOK_PAD = 20480

_DUMMY_TGT = N_NODES


@functools.lru_cache(maxsize=None)
def _make_sc_scatter(n_chunks, e_len):
  mesh = plsc.VectorSubcoreMesh(core_axis_name="c", subcore_axis_name="s",
                                num_cores=NC, num_subcores=NS)
  per_tile = n_chunks * CHUNK

  @functools.partial(
      pl.kernel,
      out_type=jax.ShapeDtypeStruct((NC, NODE_P, HALF), jnp.float32),
      mesh=mesh,
      scratch_types=[
          pltpu.VMEM((CHUNK,), jnp.int32),
          pltpu.VMEM((CHUNK,), jnp.int32),
          pltpu.VMEM((CHUNK, HALF), jnp.float32),
          pltpu.VMEM_SHARED((NODE_P, HALF), jnp.float32),
          pltpu.SemaphoreType.DMA,
      ],
  )
  def k(table_hbm, src_hbm, tgt_hbm, zero_hbm, out_hbm,
        src_v, tgt_v, rows_v, acc_sh, sem):
    c = lax.axis_index("c")
    s = lax.axis_index("s")
    z0 = s * ZROWS
    pltpu.sync_copy(zero_hbm.at[pl.ds(z0, ZROWS)], acc_sh.at[pl.ds(z0, ZROWS)])
    plsc.subcore_barrier()

    base = c * e_len + s * per_tile

    def chunk_body(i, carry):
      off = base + i * CHUNK
      toff = s * per_tile + i * CHUNK
      pltpu.sync_copy(src_hbm.at[pl.ds(off, CHUNK)], src_v)
      pltpu.sync_copy(tgt_hbm.at[pl.ds(toff, CHUNK)], tgt_v)
      pltpu.async_copy(table_hbm.at[src_v], rows_v, sem).wait()
      pltpu.sync_copy(rows_v, acc_sh.at[tgt_v], add=True)
      return carry

    lax.fori_loop(0, n_chunks, chunk_body, 0)
    plsc.subcore_barrier()
    pltpu.sync_copy(acc_sh.at[pl.ds(z0, ZROWS)],
                    out_hbm.at[c, pl.ds(z0, ZROWS)])

  return k


def _sc_scatter_edges(*args):
  return _make_sc_scatter(E_PAD // (NS * CHUNK), E_PAD)(*args)


def _sc_scatter_tokens(*args):
  return _make_sc_scatter(TOK_PAD // (NS * CHUNK), TOK_PAD)(*args)


_BR = NODE_P // 16
_NRB = NODE_P // _BR


def _mm_body(s_ref, w_ref, b_ref, o_ref):
  t = pl.program_id(1)
  ch = pl.program_id(2)
  x = jnp.concatenate([s_ref[0], s_ref[1]], axis=1)
  w = w_ref[ch, t]
  o_ref[0, 0] = jnp.dot(x, w, preferred_element_type=jnp.float32) + b_ref[ch, t]


def _msg_transform(states_h, w3, b3):
  return pl.pallas_call(
      _mm_body,
      grid=(_NRB, N_TYPES, NC),
      in_specs=[
          pl.BlockSpec((NC, _BR, HALF), lambda rb, t, ch: (0, rb, 0)),
          pl.BlockSpec((NC, N_TYPES, HIDDEN, HALF), lambda rb, t, ch: (0, 0, 0, 0)),
          pl.BlockSpec((NC, N_TYPES, HALF), lambda rb, t, ch: (0, 0, 0)),
      ],
      out_specs=pl.BlockSpec((1, 1, _BR, HALF), lambda rb, t, ch: (ch, t, rb, 0)),
      out_shape=jax.ShapeDtypeStruct((NC, N_TYPES, NODE_P, HALF), jnp.float32),
  )(states_h, w3, b3)


def _gru_body(a_ref, s_ref, gk_ref, gb_ref, ck_ref, cb_ref, o_ref):
  a = jnp.concatenate([a_ref[0], a_ref[1]], axis=1)
  st = jnp.concatenate([s_ref[0], s_ref[1]], axis=1)
  gi = jnp.concatenate([a, st], axis=1)
  gates = jax.nn.sigmoid(
      jnp.dot(gi, gk_ref[...], preferred_element_type=jnp.float32) + gb_ref[0])
  r = gates[:, :HIDDEN]
  u = gates[:, HIDDEN:]
  ci = jnp.concatenate([a, r * st], axis=1)
  cand = jnp.tanh(
      jnp.dot(ci, ck_ref[...], preferred_element_type=jnp.float32) + cb_ref[0])
  new = u * st + (1.0 - u) * cand
  o_ref[0] = new[:, :HALF]
  o_ref[1] = new[:, HALF:]


def _gru(agg_h, states_h, gk, gb, ck, cb):
  blk = pl.BlockSpec((NC, _BR, HALF), lambda rb: (0, rb, 0))
  return pl.pallas_call(
      _gru_body,
      grid=(_NRB,),
      in_specs=[
          blk, blk,
          pl.BlockSpec((2 * HIDDEN, 2 * HIDDEN), lambda rb: (0, 0)),
          pl.BlockSpec((1, 2 * HIDDEN), lambda rb: (0, 0)),
          pl.BlockSpec((2 * HIDDEN, HIDDEN), lambda rb: (0, 0)),
          pl.BlockSpec((1, HIDDEN), lambda rb: (0, 0)),
      ],
      out_specs=blk,
      out_shape=jax.ShapeDtypeStruct((NC, NODE_P, HALF), jnp.float32),
  )(agg_h, states_h, gk, gb, ck, cb)


def kernel(node_indices, node_segment_ids, edge_sources, edge_targets,
           embedding, type_weights, type_biases,
           gru_gate_kernel, gru_gate_bias, gru_cand_kernel, gru_cand_bias):
  i32 = jnp.int32
  emb_flat = jnp.stack([embedding[:, :HALF], embedding[:, HALF:]]).reshape(2 * VOCAB, HALF)

  src_tok = jnp.concatenate(
      [node_indices.astype(i32), jnp.zeros((TOK_PAD - N_TOKENS,), i32)])
  src2_tok = jnp.concatenate([src_tok, src_tok + VOCAB])
  tgt_tok = jnp.concatenate(
      [node_segment_ids.astype(i32),
       jnp.full((TOK_PAD - N_TOKENS,), _DUMMY_TGT, i32)])

  src_e = (edge_sources.astype(i32)
           + (jnp.arange(N_TYPES, dtype=i32) * NODE_P)[:, None]).reshape(-1)
  src_e = jnp.concatenate([src_e, jnp.zeros((E_PAD - N_TYPES * EDGES_PER_TYPE,), i32)])
  src2_e = jnp.concatenate([src_e, src_e + N_TYPES * NODE_P])
  tgt_e = jnp.concatenate(
      [edge_targets.astype(i32).reshape(-1),
       jnp.full((E_PAD - N_TYPES * EDGES_PER_TYPE,), _DUMMY_TGT, i32)])

  zero_sp = jnp.zeros((NODE_P, HALF), jnp.float32)

  states_h = _sc_scatter_tokens(emb_flat, src2_tok, tgt_tok, zero_sp)

  for layer, steps in enumerate(TIME_STEPS):
    w3 = type_weights[layer].reshape(N_TYPES, HIDDEN, NC, HALF).transpose(2, 0, 1, 3)
    b3 = type_biases[layer].reshape(N_TYPES, NC, HALF).transpose(1, 0, 2)
    gk = gru_gate_kernel[layer]
    gb = gru_gate_bias[layer].reshape(1, 2 * HIDDEN)
    ck = gru_cand_kernel[layer]
    cb = gru_cand_bias[layer].reshape(1, HIDDEN)
    for _ in range(steps):
      t_tab = _msg_transform(states_h, w3, b3).reshape(2 * N_TYPES * NODE_P, HALF)
      agg_h = _sc_scatter_edges(t_tab, src2_e, tgt_e, zero_sp)
      states_h = _gru(agg_h, states_h, gk, gb, ck, cb)

  return jnp.concatenate([states_h[0], states_h[1]], axis=1)[:N_NODES]
